# baseline (speedup 1.0000x reference)
import functools

import jax
import jax.numpy as jnp
from jax import lax
from jax.experimental import pallas as pl
from jax.experimental.pallas import tpu as pltpu
from jax.experimental.pallas import tpu_sc as plsc

DIM = 64
BATCH = 4096
SEQ = 200
TOTAL = BATCH * SEQ
NUM_CORES = 2
NUM_SUBCORES = 16
NUM_WORKERS = NUM_CORES * NUM_SUBCORES
PER_WORKER = TOTAL // NUM_WORKERS
CHUNK = 128
NCHUNK = PER_WORKER // CHUNK


def _build():
    mesh = plsc.VectorSubcoreMesh(core_axis_name="c", subcore_axis_name="s")

    @functools.partial(
        pl.kernel,
        mesh=mesh,
        out_type=jax.ShapeDtypeStruct((TOTAL, DIM), jnp.float32),
        scratch_types=[
            pltpu.VMEM((NCHUNK, CHUNK), jnp.int32),
            pltpu.VMEM((CHUNK, DIM), jnp.float32),
            pltpu.SemaphoreType.DMA,
        ],
        compiler_params=pltpu.CompilerParams(use_tc_tiling_on_sc=False),
    )
    def emb(idx_hbm, w_hbm, out_hbm, idx_v, rows_v, sem):
        wid = lax.axis_index("s") * NUM_CORES + lax.axis_index("c")
        pltpu.sync_copy(idx_hbm.at[wid], idx_v)
        base = wid * PER_WORKER

        def body(j, carry):
            pltpu.async_copy(w_hbm.at[idx_v.at[j]], rows_v, sem).wait()
            pltpu.sync_copy(rows_v, out_hbm.at[pl.ds(base + j * CHUNK, CHUNK)])
            return carry

        lax.fori_loop(0, NCHUNK, body, 0)

    return emb


_emb = _build()


def kernel(token_ids, weight):
    idx = token_ids.reshape(NUM_WORKERS, NCHUNK, CHUNK)
    out = _emb(idx, weight)
    return out.reshape(BATCH, SEQ, DIM)


# speedup vs baseline: 1.1154x; 1.1154x over previous
import functools

import jax
import jax.numpy as jnp
from jax import lax
from jax.experimental import pallas as pl
from jax.experimental.pallas import tpu as pltpu
from jax.experimental.pallas import tpu_sc as plsc

DIM = 64
BATCH = 4096
SEQ = 200
TOTAL = BATCH * SEQ
NUM_CORES = 2
NUM_SUBCORES = 16
NUM_WORKERS = NUM_CORES * NUM_SUBCORES
PER_WORKER = TOTAL // NUM_WORKERS
CHUNK = 128
NCHUNK = PER_WORKER // CHUNK
NBUF = 8
NGROUP = NCHUNK // NBUF


def _build():
    mesh = plsc.VectorSubcoreMesh(core_axis_name="c", subcore_axis_name="s")

    @functools.partial(
        pl.kernel,
        mesh=mesh,
        out_type=jax.ShapeDtypeStruct((TOTAL, DIM), jnp.float32),
        scratch_types=[
            pltpu.VMEM((NCHUNK, CHUNK), jnp.int32),
            pltpu.VMEM((NBUF, CHUNK, DIM), jnp.float32),
            pltpu.SemaphoreType.DMA((NBUF,)),
            pltpu.SemaphoreType.DMA((NBUF,)),
        ],
        compiler_params=pltpu.CompilerParams(use_tc_tiling_on_sc=False),
    )
    def emb(idx_hbm, w_hbm, out_hbm, idx_v, rows_v, gsem, osem):
        wid = lax.axis_index("s") * NUM_CORES + lax.axis_index("c")
        pltpu.sync_copy(idx_hbm.at[wid], idx_v)
        base = wid * PER_WORKER

        def fire_gather(c, b):
            pltpu.async_copy(w_hbm.at[idx_v.at[c]], rows_v.at[b], gsem.at[b])

        def drain_gather(c, b):
            pltpu.make_async_copy(
                w_hbm.at[idx_v.at[c]], rows_v.at[b], gsem.at[b]).wait()

        def out_slice(c):
            return out_hbm.at[pl.ds(base + c * CHUNK, CHUNK)]

        def fire_out(c, b):
            pltpu.async_copy(rows_v.at[b], out_slice(c), osem.at[b])

        def drain_out(c, b):
            pltpu.make_async_copy(rows_v.at[b], out_slice(c), osem.at[b]).wait()

        for b in range(NBUF):
            fire_gather(b, b)

        def body(g, carry):
            c0 = g * NBUF
            for b in range(NBUF):
                c = c0 + b
                drain_gather(c, b)
                fire_out(c, b)

                @pl.when(c + NBUF < NCHUNK)
                def _():
                    drain_out(c, b)
                    fire_gather(c + NBUF, b)

            return carry

        lax.fori_loop(0, NGROUP, body, 0)

        for b in range(NBUF):
            drain_out(NCHUNK - NBUF + b, b)

    return emb


_emb = _build()


def kernel(token_ids, weight):
    idx = token_ids.reshape(NUM_WORKERS, NCHUNK, CHUNK)
    out = _emb(idx, weight)
    return out.reshape(BATCH, SEQ, DIM)
